# initial kernel scaffold (unmeasured)
import jax
import jax.numpy as jnp
from jax import lax
from jax.experimental import pallas as pl
from jax.experimental.pallas import tpu as pltpu

N_DEV = 16
M = 4096
K_SHARD = 256
N = 2048
CHUNK = M // N_DEV


def kernel(x, w_mat):
    def body(x_ref, w_ref, out_ref, comm_ref, wbf_ref, amax_ref,
             hop_send, hop_recv, amax_send, amax_recv):
        me = lax.axis_index("i")
        left = lax.rem(me + N_DEV - 1, N_DEV)
        right = lax.rem(me + 1, N_DEV)

        barrier_sem = pltpu.get_barrier_semaphore()
        for nbr in (left, right):
            pl.semaphore_signal(
                barrier_sem, inc=1,
                device_id=(nbr,), device_id_type=pl.DeviceIdType.MESH,
            )
        pl.semaphore_wait(barrier_sem, 2)

        wbf_ref[...] = w_ref[...].astype(jnp.bfloat16)

        def partial(c):
            xs = x_ref[pl.ds(c * CHUNK, CHUNK), :].astype(jnp.bfloat16)
            return jax.lax.dot_general(
                xs, wbf_ref[...],
                dimension_numbers=(((1,), (0,)), ((), ())),
                preferred_element_type=jnp.float32,
            )

        comm_ref[0] = partial(lax.rem(me + N_DEV - 1, N_DEV)).astype(jnp.bfloat16)
        for h in range(N_DEV - 1):
            rdma = pltpu.make_async_remote_copy(
                src_ref=comm_ref.at[h],
                dst_ref=comm_ref.at[h + 1],
                send_sem=hop_send.at[h],
                recv_sem=hop_recv.at[h],
                device_id=(right,),
                device_id_type=pl.DeviceIdType.MESH,
            )
            rdma.start()
            c = lax.rem(me + 2 * N_DEV - 2 - h, N_DEV)
            p = partial(c)
            rdma.wait()
            acc = p + comm_ref[h + 1].astype(jnp.float32)
            if h < N_DEV - 2:
                comm_ref[h + 1] = acc.astype(jnp.bfloat16)
            else:
                out_ref[...] = acc

        local_amax = jnp.max(jnp.abs(out_ref[...]))
        amax_ref[0, :] = jnp.full((128,), local_amax, jnp.float32)
        sends = []
        for off in range(1, N_DEV):
            tgt = lax.rem(me + off, N_DEV)
            snd = pltpu.make_async_remote_copy(
                src_ref=amax_ref.at[pl.ds(0, 1)],
                dst_ref=amax_ref.at[pl.ds(N_DEV - off, 1)],
                send_sem=amax_send.at[off],
                recv_sem=amax_recv.at[N_DEV - off],
                device_id=(tgt,),
                device_id_type=pl.DeviceIdType.MESH,
            )
            snd.start()
            sends.append(snd)
        for r in range(1, N_DEV):
            rcv = pltpu.make_async_remote_copy(
                src_ref=amax_ref.at[pl.ds(0, 1)],
                dst_ref=amax_ref.at[pl.ds(r, 1)],
                send_sem=amax_send.at[0],
                recv_sem=amax_recv.at[r],
                device_id=(left,),
                device_id_type=pl.DeviceIdType.MESH,
            )
            rcv.wait_recv()

        gmax = jnp.max(amax_ref[...])
        scale = gmax / 448.0
        q = jnp.clip(out_ref[...] / scale, -448.0, 448.0)
        q = q.astype(jnp.float8_e4m3fn)
        out_ref[...] = q.astype(jnp.float32) * scale

        for snd in sends:
            snd.wait_send()

    return pl.pallas_call(
        body,
        out_shape=jax.ShapeDtypeStruct((CHUNK, N), jnp.float32),
        in_specs=[
            pl.BlockSpec(memory_space=pltpu.VMEM),
            pl.BlockSpec(memory_space=pltpu.VMEM),
        ],
        out_specs=pl.BlockSpec(memory_space=pltpu.VMEM),
        scratch_shapes=[
            pltpu.VMEM((N_DEV, CHUNK, N), jnp.bfloat16),
            pltpu.VMEM((K_SHARD, N), jnp.bfloat16),
            pltpu.VMEM((N_DEV, 128), jnp.float32),
            pltpu.SemaphoreType.DMA((N_DEV - 1,)),
            pltpu.SemaphoreType.DMA((N_DEV - 1,)),
            pltpu.SemaphoreType.DMA((N_DEV,)),
            pltpu.SemaphoreType.DMA((N_DEV,)),
        ],
        compiler_params=pltpu.CompilerParams(collective_id=0),
    )(x, w_mat)


# baseline (device time: 200016 ns/iter reference)
import jax
import jax.numpy as jnp
from jax import lax
from jax.experimental import pallas as pl
from jax.experimental.pallas import tpu as pltpu

N_DEV = 16
M = 4096
K_SHARD = 256
N = 2048
CHUNK = M // N_DEV
QUARTER = M // 4


def kernel(x, w_mat):
    def body(x_ref, w_ref, out_ref, comm1_ref, colsum_ref, comm2_ref,
             wbf_ref, amax_ref, s1_send, s1_recv, s2_send, s2_recv,
             amax_send, amax_recv):
        me = lax.axis_index("i")
        p = me // 4
        q = lax.rem(me, 4)
        up = lax.rem(me + 4, N_DEV)
        down = lax.rem(me + N_DEV - 4, N_DEV)
        rightp = 4 * p + lax.rem(q + 1, 4)
        leftp = 4 * p + lax.rem(q + 3, 4)

        barrier_sem = pltpu.get_barrier_semaphore()
        for nbr in (up, down, rightp, leftp):
            pl.semaphore_signal(
                barrier_sem, inc=1,
                device_id=(nbr,), device_id_type=pl.DeviceIdType.MESH,
            )
        pl.semaphore_wait(barrier_sem, 4)

        wbf_ref[...] = w_ref[...].astype(jnp.bfloat16)

        def partial_quarter(c):
            xs = x_ref[pl.ds(c * QUARTER, QUARTER), :].astype(jnp.bfloat16)
            return jax.lax.dot_general(
                xs, wbf_ref[...],
                dimension_numbers=(((1,), (0,)), ((), ())),
                preferred_element_type=jnp.float32,
            )

        comm1_ref[0] = partial_quarter(lax.rem(p + 3, 4)).astype(jnp.bfloat16)
        for s in range(3):
            rdma = pltpu.make_async_remote_copy(
                src_ref=comm1_ref.at[s],
                dst_ref=comm1_ref.at[s + 1],
                send_sem=s1_send.at[s],
                recv_sem=s1_recv.at[s],
                device_id=(up,),
                device_id_type=pl.DeviceIdType.MESH,
            )
            rdma.start()
            c = lax.rem(p + 6 - s, 4)
            part = partial_quarter(c)
            rdma.wait()
            acc = part + comm1_ref[s + 1].astype(jnp.float32)
            if s < 2:
                comm1_ref[s + 1] = acc.astype(jnp.bfloat16)
            else:
                colsum_ref[...] = acc

        def colsum_slice(j):
            return colsum_ref[pl.ds(j * CHUNK, CHUNK), :]

        comm2_ref[0] = colsum_slice(lax.rem(q + 3, 4)).astype(jnp.bfloat16)
        for s in range(3):
            rdma = pltpu.make_async_remote_copy(
                src_ref=comm2_ref.at[s],
                dst_ref=comm2_ref.at[s + 1],
                send_sem=s2_send.at[s],
                recv_sem=s2_recv.at[s],
                device_id=(rightp,),
                device_id_type=pl.DeviceIdType.MESH,
            )
            rdma.start()
            rdma.wait()
            j = lax.rem(q + 6 - s, 4)
            acc = colsum_slice(j) + comm2_ref[s + 1].astype(jnp.float32)
            if s < 2:
                comm2_ref[s + 1] = acc.astype(jnp.bfloat16)
            else:
                out_ref[...] = acc

        local_amax = jnp.max(jnp.abs(out_ref[...]))
        amax_ref[0, :] = jnp.full((128,), local_amax, jnp.float32)
        sends = []
        for off in range(1, N_DEV):
            tgt = lax.rem(me + off, N_DEV)
            snd = pltpu.make_async_remote_copy(
                src_ref=amax_ref.at[pl.ds(0, 1)],
                dst_ref=amax_ref.at[pl.ds(N_DEV - off, 1)],
                send_sem=amax_send.at[off],
                recv_sem=amax_recv.at[N_DEV - off],
                device_id=(tgt,),
                device_id_type=pl.DeviceIdType.MESH,
            )
            snd.start()
            sends.append(snd)
        for r in range(1, N_DEV):
            rcv = pltpu.make_async_remote_copy(
                src_ref=amax_ref.at[pl.ds(0, 1)],
                dst_ref=amax_ref.at[pl.ds(r, 1)],
                send_sem=amax_send.at[0],
                recv_sem=amax_recv.at[r],
                device_id=(down,),
                device_id_type=pl.DeviceIdType.MESH,
            )
            rcv.wait_recv()

        gmax = jnp.max(amax_ref[...])
        scale = gmax / 448.0
        qv = jnp.clip(out_ref[...] / scale, -448.0, 448.0)
        qv = qv.astype(jnp.float8_e4m3fn)
        out_ref[...] = qv.astype(jnp.float32) * scale

        for snd in sends:
            snd.wait_send()

    return pl.pallas_call(
        body,
        out_shape=jax.ShapeDtypeStruct((CHUNK, N), jnp.float32),
        in_specs=[
            pl.BlockSpec(memory_space=pltpu.VMEM),
            pl.BlockSpec(memory_space=pltpu.VMEM),
        ],
        out_specs=pl.BlockSpec(memory_space=pltpu.VMEM),
        scratch_shapes=[
            pltpu.VMEM((4, QUARTER, N), jnp.bfloat16),
            pltpu.VMEM((QUARTER, N), jnp.float32),
            pltpu.VMEM((4, CHUNK, N), jnp.bfloat16),
            pltpu.VMEM((K_SHARD, N), jnp.bfloat16),
            pltpu.VMEM((N_DEV, 128), jnp.float32),
            pltpu.SemaphoreType.DMA((3,)),
            pltpu.SemaphoreType.DMA((3,)),
            pltpu.SemaphoreType.DMA((3,)),
            pltpu.SemaphoreType.DMA((3,)),
            pltpu.SemaphoreType.DMA((N_DEV,)),
            pltpu.SemaphoreType.DMA((N_DEV,)),
        ],
        compiler_params=pltpu.CompilerParams(collective_id=0),
    )(x, w_mat)


# device time: 183283 ns/iter; 1.0913x vs baseline; 1.0913x over previous
import jax
import jax.numpy as jnp
from jax import lax
from jax.experimental import pallas as pl
from jax.experimental.pallas import tpu as pltpu

N_DEV = 16
M = 4096
K_SHARD = 256
N = 2048
CHUNK = M // N_DEV
QUARTER = M // 4
H1 = QUARTER // 2
H2 = CHUNK // 2


def kernel(x, w_mat):
    def body(x_ref, w_ref, out_ref,
             c1cw_ref, c1ccw_ref, colsum_ref, c2cw_ref, c2ccw_ref,
             wbf_ref, amax_ref,
             s1cw_s, s1cw_r, s1ccw_s, s1ccw_r,
             s2cw_s, s2cw_r, s2ccw_s, s2ccw_r,
             amax_send, amax_recv):
        me = lax.axis_index("i")
        p = me // 4
        q = lax.rem(me, 4)
        up = lax.rem(me + 4, N_DEV)
        down = lax.rem(me + N_DEV - 4, N_DEV)
        rightp = 4 * p + lax.rem(q + 1, 4)
        leftp = 4 * p + lax.rem(q + 3, 4)

        barrier_sem = pltpu.get_barrier_semaphore()
        for nbr in (up, down, rightp, leftp):
            pl.semaphore_signal(
                barrier_sem, inc=1,
                device_id=(nbr,), device_id_type=pl.DeviceIdType.MESH,
            )
        pl.semaphore_wait(barrier_sem, 4)

        wbf_ref[...] = w_ref[...].astype(jnp.bfloat16)

        def partial_rows(start, nrows):
            xs = x_ref[pl.ds(start, nrows), :].astype(jnp.bfloat16)
            return jax.lax.dot_general(
                xs, wbf_ref[...],
                dimension_numbers=(((1,), (0,)), ((), ())),
                preferred_element_type=jnp.float32,
            )

        def ring_hop(s, src_ref, sem_s, sem_r, tgt):
            rdma = pltpu.make_async_remote_copy(
                src_ref=src_ref.at[s],
                dst_ref=src_ref.at[s + 1],
                send_sem=sem_s.at[s],
                recv_sem=sem_r.at[s],
                device_id=(tgt,),
                device_id_type=pl.DeviceIdType.MESH,
            )
            rdma.start()
            return rdma

        c1cw_ref[0] = partial_rows(lax.rem(p + 3, 4) * QUARTER, H1
                                   ).astype(jnp.bfloat16)
        c1ccw_ref[0] = partial_rows(lax.rem(p + 1, 4) * QUARTER + H1, H1
                                    ).astype(jnp.bfloat16)
        for s in range(3):
            rcw = ring_hop(s, c1cw_ref, s1cw_s, s1cw_r, up)
            rccw = ring_hop(s, c1ccw_ref, s1ccw_s, s1ccw_r, down)
            c_cw = lax.rem(p + 6 - s, 4)
            c_ccw = lax.rem(p + 2 + s, 4)
            pt = partial_rows(c_cw * QUARTER, H1)
            pb = partial_rows(c_ccw * QUARTER + H1, H1)
            rcw.wait()
            acc = pt + c1cw_ref[s + 1].astype(jnp.float32)
            if s < 2:
                c1cw_ref[s + 1] = acc.astype(jnp.bfloat16)
            else:
                colsum_ref[pl.ds(0, H1), :] = acc
            rccw.wait()
            acc = pb + c1ccw_ref[s + 1].astype(jnp.float32)
            if s < 2:
                c1ccw_ref[s + 1] = acc.astype(jnp.bfloat16)
            else:
                colsum_ref[pl.ds(H1, H1), :] = acc

        def colsum_half(j, half):
            return colsum_ref[pl.ds(j * CHUNK + half * H2, H2), :]

        c2cw_ref[0] = colsum_half(lax.rem(q + 3, 4), 0).astype(jnp.bfloat16)
        c2ccw_ref[0] = colsum_half(lax.rem(q + 1, 4), 1).astype(jnp.bfloat16)
        for s in range(3):
            rcw = ring_hop(s, c2cw_ref, s2cw_s, s2cw_r, rightp)
            rccw = ring_hop(s, c2ccw_ref, s2ccw_s, s2ccw_r, leftp)
            j_cw = lax.rem(q + 6 - s, 4)
            j_ccw = lax.rem(q + 2 + s, 4)
            rcw.wait()
            acc = colsum_half(j_cw, 0) + c2cw_ref[s + 1].astype(jnp.float32)
            if s < 2:
                c2cw_ref[s + 1] = acc.astype(jnp.bfloat16)
            else:
                out_ref[pl.ds(0, H2), :] = acc
            rccw.wait()
            acc = colsum_half(j_ccw, 1) + c2ccw_ref[s + 1].astype(jnp.float32)
            if s < 2:
                c2ccw_ref[s + 1] = acc.astype(jnp.bfloat16)
            else:
                out_ref[pl.ds(H2, H2), :] = acc

        local_amax = jnp.max(jnp.abs(out_ref[...]))
        amax_ref[0, :] = jnp.full((128,), local_amax, jnp.float32)
        sends = []
        for off in range(1, N_DEV):
            tgt = lax.rem(me + off, N_DEV)
            snd = pltpu.make_async_remote_copy(
                src_ref=amax_ref.at[pl.ds(0, 1)],
                dst_ref=amax_ref.at[pl.ds(N_DEV - off, 1)],
                send_sem=amax_send.at[off],
                recv_sem=amax_recv.at[N_DEV - off],
                device_id=(tgt,),
                device_id_type=pl.DeviceIdType.MESH,
            )
            snd.start()
            sends.append(snd)
        for r in range(1, N_DEV):
            rcv = pltpu.make_async_remote_copy(
                src_ref=amax_ref.at[pl.ds(0, 1)],
                dst_ref=amax_ref.at[pl.ds(r, 1)],
                send_sem=amax_send.at[0],
                recv_sem=amax_recv.at[r],
                device_id=(down,),
                device_id_type=pl.DeviceIdType.MESH,
            )
            rcv.wait_recv()

        gmax = jnp.max(amax_ref[...])
        scale = gmax / 448.0
        qv = jnp.clip(out_ref[...] / scale, -448.0, 448.0)
        qv = qv.astype(jnp.float8_e4m3fn)
        out_ref[...] = qv.astype(jnp.float32) * scale

        for snd in sends:
            snd.wait_send()

    return pl.pallas_call(
        body,
        out_shape=jax.ShapeDtypeStruct((CHUNK, N), jnp.float32),
        in_specs=[
            pl.BlockSpec(memory_space=pltpu.VMEM),
            pl.BlockSpec(memory_space=pltpu.VMEM),
        ],
        out_specs=pl.BlockSpec(memory_space=pltpu.VMEM),
        scratch_shapes=[
            pltpu.VMEM((4, H1, N), jnp.bfloat16),
            pltpu.VMEM((4, H1, N), jnp.bfloat16),
            pltpu.VMEM((QUARTER, N), jnp.float32),
            pltpu.VMEM((4, H2, N), jnp.bfloat16),
            pltpu.VMEM((4, H2, N), jnp.bfloat16),
            pltpu.VMEM((K_SHARD, N), jnp.bfloat16),
            pltpu.VMEM((N_DEV, 128), jnp.float32),
            pltpu.SemaphoreType.DMA((3,)),
            pltpu.SemaphoreType.DMA((3,)),
            pltpu.SemaphoreType.DMA((3,)),
            pltpu.SemaphoreType.DMA((3,)),
            pltpu.SemaphoreType.DMA((3,)),
            pltpu.SemaphoreType.DMA((3,)),
            pltpu.SemaphoreType.DMA((3,)),
            pltpu.SemaphoreType.DMA((3,)),
            pltpu.SemaphoreType.DMA((N_DEV,)),
            pltpu.SemaphoreType.DMA((N_DEV,)),
        ],
        compiler_params=pltpu.CompilerParams(collective_id=0),
    )(x, w_mat)


# device time: 134311 ns/iter; 1.4892x vs baseline; 1.3646x over previous
import jax
import jax.numpy as jnp
from jax import lax
from jax.experimental import pallas as pl
from jax.experimental.pallas import tpu as pltpu

N_DEV = 16
M = 4096
K_SHARD = 256
N = 2048
CHUNK = M // N_DEV
H2 = CHUNK // 2


def kernel(x, w_mat):
    def body(x_ref, w_ref, out_ref,
             c1cw_ref, c1ccw_ref, psum_ref, c2cw_ref, c2ccw_ref,
             wbf_ref, amax_ref,
             s1cw_s, s1cw_r, s1ccw_s, s1ccw_r,
             s2cw_s, s2cw_r, s2ccw_s, s2ccw_r,
             amax_send, amax_recv):
        me = lax.axis_index("i")
        p = me // 4
        q = lax.rem(me, 4)
        rightp = 4 * p + lax.rem(q + 1, 4)
        leftp = 4 * p + lax.rem(q + 3, 4)
        up = lax.rem(me + 4, N_DEV)
        down = lax.rem(me + N_DEV - 4, N_DEV)

        barrier_sem = pltpu.get_barrier_semaphore()
        for nbr in (up, down, rightp, leftp):
            pl.semaphore_signal(
                barrier_sem, inc=1,
                device_id=(nbr,), device_id_type=pl.DeviceIdType.MESH,
            )
        pl.semaphore_wait(barrier_sem, 4)

        wbf_ref[...] = w_ref[...].astype(jnp.bfloat16)

        def partial_block(b, c):
            xs = x_ref[pl.ds((4 * b + c) * CHUNK, CHUNK), :].astype(
                jnp.bfloat16)
            return jax.lax.dot_general(
                xs, wbf_ref[...],
                dimension_numbers=(((1,), (0,)), ((), ())),
                preferred_element_type=jnp.float32,
            )

        def ring_hop(s, buf_ref, sem_s, sem_r, tgt):
            rdma = pltpu.make_async_remote_copy(
                src_ref=buf_ref.at[s],
                dst_ref=buf_ref.at[s + 1],
                send_sem=sem_s.at[s],
                recv_sem=sem_r.at[s],
                device_id=(tgt,),
                device_id_type=pl.DeviceIdType.MESH,
            )
            rdma.start()
            return rdma

        c0 = lax.rem(q + 3, 4)
        for b in (0, 1):
            c1cw_ref[0, b] = partial_block(b, c0).astype(jnp.bfloat16)
        c0 = lax.rem(q + 1, 4)
        for b in (2, 3):
            c1ccw_ref[0, b - 2] = partial_block(b, c0).astype(jnp.bfloat16)
        for s in range(3):
            rcw = ring_hop(s, c1cw_ref, s1cw_s, s1cw_r, rightp)
            rccw = ring_hop(s, c1ccw_ref, s1ccw_s, s1ccw_r, leftp)
            c_cw = lax.rem(q + 6 - s, 4)
            c_ccw = lax.rem(q + 2 + s, 4)
            pcw = [partial_block(b, c_cw) for b in (0, 1)]
            pccw = [partial_block(b, c_ccw) for b in (2, 3)]
            rcw.wait()
            for b in (0, 1):
                acc = pcw[b] + c1cw_ref[s + 1, b].astype(jnp.float32)
                if s < 2:
                    c1cw_ref[s + 1, b] = acc.astype(jnp.bfloat16)
                else:
                    psum_ref[pl.ds(b * CHUNK, CHUNK), :] = acc
            rccw.wait()
            for b in (2, 3):
                acc = pccw[b - 2] + c1ccw_ref[s + 1, b - 2].astype(jnp.float32)
                if s < 2:
                    c1ccw_ref[s + 1, b - 2] = acc.astype(jnp.bfloat16)
                else:
                    psum_ref[pl.ds(b * CHUNK, CHUNK), :] = acc


        def psum_half(j, half):
            return psum_ref[pl.ds(j * CHUNK + half * H2, H2), :]

        c2cw_ref[0] = psum_half(lax.rem(p + 3, 4), 0).astype(jnp.bfloat16)
        c2ccw_ref[0] = psum_half(lax.rem(p + 1, 4), 1).astype(jnp.bfloat16)
        for s in range(3):
            rcw = ring_hop(s, c2cw_ref, s2cw_s, s2cw_r, up)
            rccw = ring_hop(s, c2ccw_ref, s2ccw_s, s2ccw_r, down)
            j_cw = lax.rem(p + 6 - s, 4)
            j_ccw = lax.rem(p + 2 + s, 4)
            rcw.wait()
            acc = psum_half(j_cw, 0) + c2cw_ref[s + 1].astype(jnp.float32)
            if s < 2:
                c2cw_ref[s + 1] = acc.astype(jnp.bfloat16)
            else:
                out_ref[pl.ds(0, H2), :] = acc
            rccw.wait()
            acc = psum_half(j_ccw, 1) + c2ccw_ref[s + 1].astype(jnp.float32)
            if s < 2:
                c2ccw_ref[s + 1] = acc.astype(jnp.bfloat16)
            else:
                out_ref[pl.ds(H2, H2), :] = acc

        local_amax = jnp.max(jnp.abs(out_ref[...]))
        amax_ref[0, :] = jnp.full((128,), local_amax, jnp.float32)
        sends = []
        for off in range(1, N_DEV):
            tgt = lax.rem(me + off, N_DEV)
            snd = pltpu.make_async_remote_copy(
                src_ref=amax_ref.at[pl.ds(0, 1)],
                dst_ref=amax_ref.at[pl.ds(N_DEV - off, 1)],
                send_sem=amax_send.at[off],
                recv_sem=amax_recv.at[N_DEV - off],
                device_id=(tgt,),
                device_id_type=pl.DeviceIdType.MESH,
            )
            snd.start()
            sends.append(snd)
        for r in range(1, N_DEV):
            rcv = pltpu.make_async_remote_copy(
                src_ref=amax_ref.at[pl.ds(0, 1)],
                dst_ref=amax_ref.at[pl.ds(r, 1)],
                send_sem=amax_send.at[0],
                recv_sem=amax_recv.at[r],
                device_id=(down,),
                device_id_type=pl.DeviceIdType.MESH,
            )
            rcv.wait_recv()

        gmax = jnp.max(amax_ref[...])
        scale = gmax / 448.0
        qv = jnp.clip(out_ref[...] / scale, -448.0, 448.0)
        qv = qv.astype(jnp.float8_e4m3fn)
        out_ref[...] = qv.astype(jnp.float32) * scale

        for snd in sends:
            snd.wait_send()

    return pl.pallas_call(
        body,
        out_shape=jax.ShapeDtypeStruct((CHUNK, N), jnp.float32),
        in_specs=[
            pl.BlockSpec(memory_space=pltpu.VMEM),
            pl.BlockSpec(memory_space=pltpu.VMEM),
        ],
        out_specs=pl.BlockSpec(memory_space=pltpu.VMEM),
        scratch_shapes=[
            pltpu.VMEM((4, 2, CHUNK, N), jnp.bfloat16),
            pltpu.VMEM((4, 2, CHUNK, N), jnp.bfloat16),
            pltpu.VMEM((4 * CHUNK, N), jnp.float32),
            pltpu.VMEM((4, H2, N), jnp.bfloat16),
            pltpu.VMEM((4, H2, N), jnp.bfloat16),
            pltpu.VMEM((K_SHARD, N), jnp.bfloat16),
            pltpu.VMEM((N_DEV, 128), jnp.float32),
            pltpu.SemaphoreType.DMA((3,)),
            pltpu.SemaphoreType.DMA((3,)),
            pltpu.SemaphoreType.DMA((3,)),
            pltpu.SemaphoreType.DMA((3,)),
            pltpu.SemaphoreType.DMA((3,)),
            pltpu.SemaphoreType.DMA((3,)),
            pltpu.SemaphoreType.DMA((3,)),
            pltpu.SemaphoreType.DMA((3,)),
            pltpu.SemaphoreType.DMA((N_DEV,)),
            pltpu.SemaphoreType.DMA((N_DEV,)),
        ],
        compiler_params=pltpu.CompilerParams(collective_id=0),
    )(x, w_mat)


# device time: 123339 ns/iter; 1.6217x vs baseline; 1.0890x over previous
import jax
import jax.numpy as jnp
from jax import lax
from jax.experimental import pallas as pl
from jax.experimental.pallas import tpu as pltpu

N_DEV = 16
M = 4096
K_SHARD = 256
N = 2048
CHUNK = M // N_DEV
H = CHUNK // 2


def kernel(x, w_mat):
    def body(x_ref, w_ref, out_ref,
             cacw_ref, caccw_ref, cbcw_ref, cbccw_ref,
             psum_ref, czt_ref, czb_ref, wbf_ref, amax_ref,
             sacw_s, sacw_r, saccw_s, saccw_r,
             sbcw_s, sbcw_r, sbccw_s, sbccw_r,
             szt_s, szt_r, szb_s, szb_r,
             amax_send, amax_recv):
        me = lax.axis_index("i")
        p = me // 4
        q = lax.rem(me, 4)
        rightp = 4 * p + lax.rem(q + 1, 4)
        leftp = 4 * p + lax.rem(q + 3, 4)
        up = lax.rem(me + 4, N_DEV)
        down = lax.rem(me + N_DEV - 4, N_DEV)

        barrier_sem = pltpu.get_barrier_semaphore()
        for nbr in (up, down, rightp, leftp):
            pl.semaphore_signal(
                barrier_sem, inc=1,
                device_id=(nbr,), device_id_type=pl.DeviceIdType.MESH,
            )
        pl.semaphore_wait(barrier_sem, 4)

        wbf_ref[...] = w_ref[...].astype(jnp.bfloat16)

        def partial_half(b, c, half):
            xs = x_ref[pl.ds((4 * b + c) * CHUNK + half * H, H), :].astype(
                jnp.bfloat16)
            return jax.lax.dot_general(
                xs, wbf_ref[...],
                dimension_numbers=(((1,), (0,)), ((), ())),
                preferred_element_type=jnp.float32,
            )

        def psum_half(b, half):
            return psum_ref[pl.ds(b * CHUNK + half * H, H), :]

        def psum_store(b, half, val):
            psum_ref[pl.ds(b * CHUNK + half * H, H), :] = val

        def ring_hop(s, buf_ref, sem_s, sem_r, tgt):
            rdma = pltpu.make_async_remote_copy(
                src_ref=buf_ref.at[s],
                dst_ref=buf_ref.at[s + 1],
                send_sem=sem_s.at[s],
                recv_sem=sem_r.at[s],
                device_id=(tgt,),
                device_id_type=pl.DeviceIdType.MESH,
            )
            rdma.start()
            return rdma

        def plane_phase(half, cw_ref, ccw_ref, cw_s, cw_r, ccw_s, ccw_r):
            c0 = lax.rem(q + 3, 4)
            for b in (0, 1):
                cw_ref[0, b] = partial_half(b, c0, half).astype(jnp.bfloat16)
            c0 = lax.rem(q + 1, 4)
            for b in (2, 3):
                ccw_ref[0, b - 2] = partial_half(b, c0, half).astype(
                    jnp.bfloat16)
            for s in range(3):
                rcw = ring_hop(s, cw_ref, cw_s, cw_r, rightp)
                rccw = ring_hop(s, ccw_ref, ccw_s, ccw_r, leftp)
                c_cw = lax.rem(q + 6 - s, 4)
                c_ccw = lax.rem(q + 2 + s, 4)
                pcw = [partial_half(b, c_cw, half) for b in (0, 1)]
                pccw = [partial_half(b, c_ccw, half) for b in (2, 3)]
                rcw.wait()
                for b in (0, 1):
                    acc = pcw[b] + cw_ref[s + 1, b].astype(jnp.float32)
                    if s < 2:
                        cw_ref[s + 1, b] = acc.astype(jnp.bfloat16)
                    else:
                        psum_store(b, half, acc)
                rccw.wait()
                for b in (2, 3):
                    acc = pccw[b - 2] + ccw_ref[s + 1, b - 2].astype(
                        jnp.float32)
                    if s < 2:
                        ccw_ref[s + 1, b - 2] = acc.astype(jnp.bfloat16)
                    else:
                        psum_store(b, half, acc)

        plane_phase(0, cacw_ref, caccw_ref, sacw_s, sacw_r, saccw_s, saccw_r)

        czt_ref[0] = psum_half(lax.rem(p + 3, 4), 0).astype(jnp.bfloat16)

        c0 = lax.rem(q + 3, 4)
        for b in (0, 1):
            cbcw_ref[0, b] = partial_half(b, c0, 1).astype(jnp.bfloat16)
        c0 = lax.rem(q + 1, 4)
        for b in (2, 3):
            cbccw_ref[0, b - 2] = partial_half(b, c0, 1).astype(jnp.bfloat16)
        for s in range(3):
            rcw = ring_hop(s, cbcw_ref, sbcw_s, sbcw_r, rightp)
            rccw = ring_hop(s, cbccw_ref, sbccw_s, sbccw_r, leftp)
            rzt = ring_hop(s, czt_ref, szt_s, szt_r, up)
            c_cw = lax.rem(q + 6 - s, 4)
            c_ccw = lax.rem(q + 2 + s, 4)
            pcw = [partial_half(b, c_cw, 1) for b in (0, 1)]
            pccw = [partial_half(b, c_ccw, 1) for b in (2, 3)]
            rzt.wait()
            j = lax.rem(p + 6 - s, 4)
            acc = psum_half(j, 0) + czt_ref[s + 1].astype(jnp.float32)
            if s < 2:
                czt_ref[s + 1] = acc.astype(jnp.bfloat16)
            else:
                out_ref[pl.ds(0, H), :] = acc
            rcw.wait()
            for b in (0, 1):
                acc = pcw[b] + cbcw_ref[s + 1, b].astype(jnp.float32)
                if s < 2:
                    cbcw_ref[s + 1, b] = acc.astype(jnp.bfloat16)
                else:
                    psum_store(b, 1, acc)
            rccw.wait()
            for b in (2, 3):
                acc = pccw[b - 2] + cbccw_ref[s + 1, b - 2].astype(jnp.float32)
                if s < 2:
                    cbccw_ref[s + 1, b - 2] = acc.astype(jnp.bfloat16)
                else:
                    psum_store(b, 1, acc)

        czb_ref[0] = psum_half(lax.rem(p + 1, 4), 1).astype(jnp.bfloat16)
        for s in range(3):
            rzb = ring_hop(s, czb_ref, szb_s, szb_r, down)
            rzb.wait()
            j = lax.rem(p + 2 + s, 4)
            acc = psum_half(j, 1) + czb_ref[s + 1].astype(jnp.float32)
            if s < 2:
                czb_ref[s + 1] = acc.astype(jnp.bfloat16)
            else:
                out_ref[pl.ds(H, H), :] = acc

        local_amax = jnp.max(jnp.abs(out_ref[...]))
        amax_ref[0, :] = jnp.full((128,), local_amax, jnp.float32)
        sends = []
        for off in range(1, N_DEV):
            tgt = lax.rem(me + off, N_DEV)
            snd = pltpu.make_async_remote_copy(
                src_ref=amax_ref.at[pl.ds(0, 1)],
                dst_ref=amax_ref.at[pl.ds(N_DEV - off, 1)],
                send_sem=amax_send.at[off],
                recv_sem=amax_recv.at[N_DEV - off],
                device_id=(tgt,),
                device_id_type=pl.DeviceIdType.MESH,
            )
            snd.start()
            sends.append(snd)
        for r in range(1, N_DEV):
            rcv = pltpu.make_async_remote_copy(
                src_ref=amax_ref.at[pl.ds(0, 1)],
                dst_ref=amax_ref.at[pl.ds(r, 1)],
                send_sem=amax_send.at[0],
                recv_sem=amax_recv.at[r],
                device_id=(down,),
                device_id_type=pl.DeviceIdType.MESH,
            )
            rcv.wait_recv()

        gmax = jnp.max(amax_ref[...])
        scale = gmax / 448.0
        qv = jnp.clip(out_ref[...] / scale, -448.0, 448.0)
        qv = qv.astype(jnp.float8_e4m3fn)
        out_ref[...] = qv.astype(jnp.float32) * scale

        for snd in sends:
            snd.wait_send()

    return pl.pallas_call(
        body,
        out_shape=jax.ShapeDtypeStruct((CHUNK, N), jnp.float32),
        in_specs=[
            pl.BlockSpec(memory_space=pltpu.VMEM),
            pl.BlockSpec(memory_space=pltpu.VMEM),
        ],
        out_specs=pl.BlockSpec(memory_space=pltpu.VMEM),
        scratch_shapes=[
            pltpu.VMEM((4, 2, H, N), jnp.bfloat16),
            pltpu.VMEM((4, 2, H, N), jnp.bfloat16),
            pltpu.VMEM((4, 2, H, N), jnp.bfloat16),
            pltpu.VMEM((4, 2, H, N), jnp.bfloat16),
            pltpu.VMEM((4 * CHUNK, N), jnp.float32),
            pltpu.VMEM((4, H, N), jnp.bfloat16),
            pltpu.VMEM((4, H, N), jnp.bfloat16),
            pltpu.VMEM((K_SHARD, N), jnp.bfloat16),
            pltpu.VMEM((N_DEV, 128), jnp.float32),
            pltpu.SemaphoreType.DMA((3,)),
            pltpu.SemaphoreType.DMA((3,)),
            pltpu.SemaphoreType.DMA((3,)),
            pltpu.SemaphoreType.DMA((3,)),
            pltpu.SemaphoreType.DMA((3,)),
            pltpu.SemaphoreType.DMA((3,)),
            pltpu.SemaphoreType.DMA((3,)),
            pltpu.SemaphoreType.DMA((3,)),
            pltpu.SemaphoreType.DMA((3,)),
            pltpu.SemaphoreType.DMA((3,)),
            pltpu.SemaphoreType.DMA((3,)),
            pltpu.SemaphoreType.DMA((3,)),
            pltpu.SemaphoreType.DMA((N_DEV,)),
            pltpu.SemaphoreType.DMA((N_DEV,)),
        ],
        compiler_params=pltpu.CompilerParams(collective_id=0),
    )(x, w_mat)


# device time: 118890 ns/iter; 1.6824x vs baseline; 1.0374x over previous
import jax
import jax.numpy as jnp
from jax import lax
from jax.experimental import pallas as pl
from jax.experimental.pallas import tpu as pltpu

N_DEV = 16
M = 4096
K_SHARD = 256
N = 2048
CHUNK = M // N_DEV
H = CHUNK // 2
Q4 = CHUNK // 4


def kernel(x, w_mat):
    def body(x_ref, w_ref, out_ref,
             cacw_ref, caccw_ref, cbcw_ref, cbccw_ref,
             psum_ref, czt_ref, czbu_ref, czbd_ref, wbf_ref, amax_ref,
             sacw_s, sacw_r, saccw_s, saccw_r,
             sbcw_s, sbcw_r, sbccw_s, sbccw_r,
             szt_s, szt_r, szbu_s, szbu_r, szbd_s, szbd_r,
             amax_send, amax_recv):
        me = lax.axis_index("i")
        p = me // 4
        q = lax.rem(me, 4)
        rightp = 4 * p + lax.rem(q + 1, 4)
        leftp = 4 * p + lax.rem(q + 3, 4)
        up = lax.rem(me + 4, N_DEV)
        down = lax.rem(me + N_DEV - 4, N_DEV)

        barrier_sem = pltpu.get_barrier_semaphore()
        for nbr in (up, down, rightp, leftp):
            pl.semaphore_signal(
                barrier_sem, inc=1,
                device_id=(nbr,), device_id_type=pl.DeviceIdType.MESH,
            )
        pl.semaphore_wait(barrier_sem, 4)

        wbf_ref[...] = w_ref[...].astype(jnp.bfloat16)

        started = []

        def partial_rows(b, c, off, nrows):
            xs = x_ref[pl.ds((4 * b + c) * CHUNK + off, nrows), :].astype(
                jnp.bfloat16)
            return jax.lax.dot_general(
                xs, wbf_ref[...],
                dimension_numbers=(((1,), (0,)), ((), ())),
                preferred_element_type=jnp.float32,
            )

        def psum_rows(b, off, nrows):
            return psum_ref[pl.ds(b * CHUNK + off, nrows), :]

        def psum_store(b, off, nrows, val):
            psum_ref[pl.ds(b * CHUNK + off, nrows), :] = val

        def ring_hop(s, buf_ref, sem_s, sem_r, tgt):
            rdma = pltpu.make_async_remote_copy(
                src_ref=buf_ref.at[s],
                dst_ref=buf_ref.at[s + 1],
                send_sem=sem_s.at[s],
                recv_sem=sem_r.at[s],
                device_id=(tgt,),
                device_id_type=pl.DeviceIdType.MESH,
            )
            rdma.start()
            started.append(rdma)
            return rdma

        def plane_phase(half, cw_ref, ccw_ref, cw_s, cw_r, ccw_s, ccw_r,
                        zt_step=None):
            off = half * H
            c0 = lax.rem(q + 3, 4)
            for b in (0, 1):
                cw_ref[0, b] = partial_rows(b, c0, off, H).astype(jnp.bfloat16)
            c0 = lax.rem(q + 1, 4)
            for b in (2, 3):
                ccw_ref[0, b - 2] = partial_rows(b, c0, off, H).astype(
                    jnp.bfloat16)
            rcw = ring_hop(0, cw_ref, cw_s, cw_r, rightp)
            rccw = ring_hop(0, ccw_ref, ccw_s, ccw_r, leftp)
            for s in range(3):
                c_cw = lax.rem(q + 6 - s, 4)
                c_ccw = lax.rem(q + 2 + s, 4)
                pcw = [partial_rows(b, c_cw, off, H) for b in (0, 1)]
                pccw = [partial_rows(b, c_ccw, off, H) for b in (2, 3)]
                if zt_step is not None:
                    zt_step(s)
                rcw.wait_recv()
                for b in (0, 1):
                    acc = pcw[b] + cw_ref[s + 1, b].astype(jnp.float32)
                    if s < 2:
                        cw_ref[s + 1, b] = acc.astype(jnp.bfloat16)
                    else:
                        psum_store(b, off, H, acc)
                if s < 2:
                    rcw = ring_hop(s + 1, cw_ref, cw_s, cw_r, rightp)
                rccw.wait_recv()
                for b in (2, 3):
                    acc = pccw[b - 2] + ccw_ref[s + 1, b - 2].astype(
                        jnp.float32)
                    if s < 2:
                        ccw_ref[s + 1, b - 2] = acc.astype(jnp.bfloat16)
                    else:
                        psum_store(b, off, H, acc)
                if s < 2:
                    rccw = ring_hop(s + 1, ccw_ref, ccw_s, ccw_r, leftp)

        plane_phase(0, cacw_ref, caccw_ref, sacw_s, sacw_r, saccw_s, saccw_r)

        czt_ref[0] = psum_rows(lax.rem(p + 3, 4), 0, H).astype(jnp.bfloat16)
        zt_state = {"rdma": ring_hop(0, czt_ref, szt_s, szt_r, up)}

        def zt_step(s):
            zt_state["rdma"].wait_recv()
            j = lax.rem(p + 6 - s, 4)
            acc = psum_rows(j, 0, H) + czt_ref[s + 1].astype(jnp.float32)
            if s < 2:
                czt_ref[s + 1] = acc.astype(jnp.bfloat16)
                zt_state["rdma"] = ring_hop(s + 1, czt_ref, szt_s, szt_r, up)
            else:
                out_ref[pl.ds(0, H), :] = acc

        plane_phase(1, cbcw_ref, cbccw_ref, sbcw_s, sbcw_r, sbccw_s, sbccw_r,
                    zt_step=zt_step)

        czbu_ref[0] = psum_rows(lax.rem(p + 3, 4), H, Q4).astype(jnp.bfloat16)
        czbd_ref[0] = psum_rows(lax.rem(p + 1, 4), H + Q4, Q4).astype(
            jnp.bfloat16)
        rzu = ring_hop(0, czbu_ref, szbu_s, szbu_r, up)
        rzd = ring_hop(0, czbd_ref, szbd_s, szbd_r, down)
        amax_top = jnp.max(jnp.abs(out_ref[pl.ds(0, H), :]))
        for s in range(3):
            rzu.wait_recv()
            j = lax.rem(p + 6 - s, 4)
            acc = psum_rows(j, H, Q4) + czbu_ref[s + 1].astype(jnp.float32)
            if s < 2:
                czbu_ref[s + 1] = acc.astype(jnp.bfloat16)
                rzu = ring_hop(s + 1, czbu_ref, szbu_s, szbu_r, up)
            else:
                out_ref[pl.ds(H, Q4), :] = acc
            rzd.wait_recv()
            j = lax.rem(p + 2 + s, 4)
            acc = psum_rows(j, H + Q4, Q4) + czbd_ref[s + 1].astype(
                jnp.float32)
            if s < 2:
                czbd_ref[s + 1] = acc.astype(jnp.bfloat16)
                rzd = ring_hop(s + 1, czbd_ref, szbd_s, szbd_r, down)
            else:
                out_ref[pl.ds(H + Q4, Q4), :] = acc

        local_amax = jnp.maximum(
            amax_top, jnp.max(jnp.abs(out_ref[pl.ds(H, H), :])))
        amax_ref[0, :] = jnp.full((128,), local_amax, jnp.float32)
        for off in range(1, N_DEV):
            tgt = lax.rem(me + off, N_DEV)
            snd = pltpu.make_async_remote_copy(
                src_ref=amax_ref.at[pl.ds(0, 1)],
                dst_ref=amax_ref.at[pl.ds(N_DEV - off, 1)],
                send_sem=amax_send.at[off],
                recv_sem=amax_recv.at[N_DEV - off],
                device_id=(tgt,),
                device_id_type=pl.DeviceIdType.MESH,
            )
            snd.start()
            started.append(snd)
        for r in range(1, N_DEV):
            rcv = pltpu.make_async_remote_copy(
                src_ref=amax_ref.at[pl.ds(0, 1)],
                dst_ref=amax_ref.at[pl.ds(r, 1)],
                send_sem=amax_send.at[0],
                recv_sem=amax_recv.at[r],
                device_id=(down,),
                device_id_type=pl.DeviceIdType.MESH,
            )
            rcv.wait_recv()

        gmax = jnp.max(amax_ref[...])
        scale = gmax / 448.0
        qv = jnp.clip(out_ref[...] / scale, -448.0, 448.0)
        qv = qv.astype(jnp.float8_e4m3fn)
        out_ref[...] = qv.astype(jnp.float32) * scale

        for rdma in started:
            rdma.wait_send()

    return pl.pallas_call(
        body,
        out_shape=jax.ShapeDtypeStruct((CHUNK, N), jnp.float32),
        in_specs=[
            pl.BlockSpec(memory_space=pltpu.VMEM),
            pl.BlockSpec(memory_space=pltpu.VMEM),
        ],
        out_specs=pl.BlockSpec(memory_space=pltpu.VMEM),
        scratch_shapes=[
            pltpu.VMEM((4, 2, H, N), jnp.bfloat16),
            pltpu.VMEM((4, 2, H, N), jnp.bfloat16),
            pltpu.VMEM((4, 2, H, N), jnp.bfloat16),
            pltpu.VMEM((4, 2, H, N), jnp.bfloat16),
            pltpu.VMEM((4 * CHUNK, N), jnp.float32),
            pltpu.VMEM((4, H, N), jnp.bfloat16),
            pltpu.VMEM((4, Q4, N), jnp.bfloat16),
            pltpu.VMEM((4, Q4, N), jnp.bfloat16),
            pltpu.VMEM((K_SHARD, N), jnp.bfloat16),
            pltpu.VMEM((N_DEV, 128), jnp.float32),
            pltpu.SemaphoreType.DMA((3,)),
            pltpu.SemaphoreType.DMA((3,)),
            pltpu.SemaphoreType.DMA((3,)),
            pltpu.SemaphoreType.DMA((3,)),
            pltpu.SemaphoreType.DMA((3,)),
            pltpu.SemaphoreType.DMA((3,)),
            pltpu.SemaphoreType.DMA((3,)),
            pltpu.SemaphoreType.DMA((3,)),
            pltpu.SemaphoreType.DMA((3,)),
            pltpu.SemaphoreType.DMA((3,)),
            pltpu.SemaphoreType.DMA((3,)),
            pltpu.SemaphoreType.DMA((3,)),
            pltpu.SemaphoreType.DMA((3,)),
            pltpu.SemaphoreType.DMA((3,)),
            pltpu.SemaphoreType.DMA((N_DEV,)),
            pltpu.SemaphoreType.DMA((N_DEV,)),
        ],
        compiler_params=pltpu.CompilerParams(collective_id=0),
    )(x, w_mat)
